# baseline (speedup 1.0000x reference)
import functools

import jax
import jax.numpy as jnp
from jax import lax
from jax.experimental import pallas as pl
from jax.experimental.pallas import tpu as pltpu

_N = 10000
_F = 128
_H1 = 32
_DZ1 = 16
_HY = 16
_NH = 2
_NY = 7
_HX = 32
_AR = 0.5

_NPT = 10240
_BR = 400
_BC = 512
_INTERPRET = False


def _ident(v):
    return v


def _mm_kernel(x_ref, w_ref, o_ref, *, act):
    o_ref[:] = act(
        jnp.dot(x_ref[:], w_ref[:], preferred_element_type=jnp.float32))


def _mm(x, w, act=_ident, br=2000):
    n, k = x.shape
    m = w.shape[1]
    return pl.pallas_call(
        functools.partial(_mm_kernel, act=act),
        grid=(n // br,),
        in_specs=[
            pl.BlockSpec((br, k), lambda i: (i, 0)),
            pl.BlockSpec((k, m), lambda i: (0, 0)),
        ],
        out_specs=pl.BlockSpec((br, m), lambda i: (i, 0)),
        out_shape=jax.ShapeDtypeStruct((n, m), jnp.float32),
        interpret=_INTERPRET,
    )(x, w)


def _mm2_kernel(a_ref, wa_ref, b_ref, wb_ref, o_ref):
    o_ref[:] = (
        jnp.dot(a_ref[:], wa_ref[:], preferred_element_type=jnp.float32)
        + jnp.dot(b_ref[:], wb_ref[:], preferred_element_type=jnp.float32))


def _mm2(a, wa, b, wb, br=2000):
    n, ka = a.shape
    kb = b.shape[1]
    m = wa.shape[1]
    return pl.pallas_call(
        _mm2_kernel,
        grid=(n // br,),
        in_specs=[
            pl.BlockSpec((br, ka), lambda i: (i, 0)),
            pl.BlockSpec((ka, m), lambda i: (0, 0)),
            pl.BlockSpec((br, kb), lambda i: (i, 0)),
            pl.BlockSpec((kb, m), lambda i: (0, 0)),
        ],
        out_specs=pl.BlockSpec((br, m), lambda i: (i, 0)),
        out_shape=jax.ShapeDtypeStruct((n, m), jnp.float32),
        interpret=_INTERPRET,
    )(a, wa, b, wb)


def _z1q_kernel(mu_ref, ls_ref, eps_ref, o_ref):
    o_ref[:] = mu_ref[:] + eps_ref[:] * jnp.exp(ls_ref[:])


def _z1q(mu, ls, eps, br=2000):
    n, m = mu.shape
    spec = pl.BlockSpec((br, m), lambda i: (i, 0))
    return pl.pallas_call(
        _z1q_kernel,
        grid=(n // br,),
        in_specs=[spec, spec, spec],
        out_specs=spec,
        out_shape=jax.ShapeDtypeStruct((n, m), jnp.float32),
        interpret=_INTERPRET,
    )(mu, ls, eps)


def _finalize_kernel(num_ref, den_ref, o_ref, *, act):
    o_ref[:] = act(num_ref[:] / den_ref[:])


def _att_finalize(num, den, act=_ident, br=2000):
    n, m = num.shape
    return pl.pallas_call(
        functools.partial(_finalize_kernel, act=act),
        grid=(n // br,),
        in_specs=[
            pl.BlockSpec((br, m), lambda i: (i, 0)),
            pl.BlockSpec((br, 1), lambda i: (i, 0)),
        ],
        out_specs=pl.BlockSpec((br, m), lambda i: (i, 0)),
        out_shape=jax.ShapeDtypeStruct((n, m), jnp.float32),
        interpret=_INTERPRET,
    )(num, den)


def _elu(v):
    return jnp.where(v > 0, v, jnp.exp(v) - 1.0)


def _colmask(c):
    cols = c * _BC + lax.broadcasted_iota(jnp.int32, (1, _BC), 1)
    return (cols < _N).astype(jnp.float32)


def _dg_kernel(z_ref, zt_ref, o_ref):
    def body(c, acc):
        logits = jnp.dot(z_ref[:], zt_ref[:, pl.ds(c * _BC, _BC)],
                         preferred_element_type=jnp.float32)
        g = jax.nn.sigmoid(logits) * _colmask(c)
        return acc + jnp.sum(g, axis=1, keepdims=True)

    s = lax.fori_loop(0, _NPT // _BC, body, jnp.zeros((_BR, 1), jnp.float32))
    o_ref[:] = s ** -0.5


def _dg_pass(z1q, z1qt):
    return pl.pallas_call(
        _dg_kernel,
        grid=(_N // _BR,),
        in_specs=[
            pl.BlockSpec((_BR, _DZ1), lambda i: (i, 0)),
            pl.BlockSpec((_DZ1, _NPT), lambda i: (0, 0)),
        ],
        out_specs=pl.BlockSpec((_BR, 1), lambda i: (i, 0)),
        out_shape=jax.ShapeDtypeStruct((_N, 1), jnp.float32),
        interpret=_INTERPRET,
    )(z1q, z1qt)


def _gmat_kernel(z_ref, zt_ref, vt_ref, dg_ref, o_ref, *, m, act, res_ref=None):
    def body(c, acc):
        logits = jnp.dot(z_ref[:], zt_ref[:, pl.ds(c * _BC, _BC)],
                         preferred_element_type=jnp.float32)
        g = jax.nn.sigmoid(logits) * _colmask(c)
        vchunk = vt_ref[:, pl.ds(c * _BC, _BC)]
        return acc + lax.dot_general(
            g, vchunk, (((1,), (1,)), ((), ())),
            preferred_element_type=jnp.float32)

    acc = lax.fori_loop(0, _NPT // _BC, body, jnp.zeros((_BR, m), jnp.float32))
    out = dg_ref[:] * acc
    if res_ref is not None:
        out = 0.5 * res_ref[:] + 0.5 * out
    o_ref[:] = act(out)


def _gmat_pass(z1q, z1qt, vt, dg, act=_ident, residual=None):
    m = vt.shape[0]
    in_specs = [
        pl.BlockSpec((_BR, _DZ1), lambda i: (i, 0)),
        pl.BlockSpec((_DZ1, _NPT), lambda i: (0, 0)),
        pl.BlockSpec((m, _NPT), lambda i: (0, 0)),
        pl.BlockSpec((_BR, 1), lambda i: (i, 0)),
    ]
    args = [z1q, z1qt, vt, dg]
    if residual is not None:
        in_specs.append(pl.BlockSpec((_BR, m), lambda i: (i, 0)))
        args.append(residual)

        def kern(z, zt, vt_, dg_, res, o):
            _gmat_kernel(z, zt, vt_, dg_, o, m=m, act=act, res_ref=res)
    else:
        def kern(z, zt, vt_, dg_, o):
            _gmat_kernel(z, zt, vt_, dg_, o, m=m, act=act)

    return pl.pallas_call(
        kern,
        grid=(_N // _BR,),
        in_specs=in_specs,
        out_specs=pl.BlockSpec((_BR, m), lambda i: (i, 0)),
        out_shape=jax.ShapeDtypeStruct((_N, m), jnp.float32),
        interpret=_INTERPRET,
    )(*args)


def _recon_kernel(e_ref, et_ref, o_ref):
    o_ref[:] = jnp.dot(e_ref[:], et_ref[:],
                       preferred_element_type=jnp.float32)


def _recon_pass(emb, embt):
    return pl.pallas_call(
        _recon_kernel,
        grid=(_N // _BR, _NPT // _BC),
        in_specs=[
            pl.BlockSpec((_BR, _DZ1), lambda i, j: (i, 0)),
            pl.BlockSpec((_DZ1, _BC), lambda i, j: (0, j)),
        ],
        out_specs=pl.BlockSpec((_BR, _BC), lambda i, j: (i, j)),
        out_shape=jax.ShapeDtypeStruct((_N, _N), jnp.float32),
        interpret=_INTERPRET,
    )(emb, embt)


def _padT(v):
    return jnp.pad(v, ((0, _NPT - _N), (0, 0))).T


def _edge_list(edge_index):
    src = edge_index[0].astype(jnp.int32)
    dst = edge_index[1].astype(jnp.int32)
    ar = jnp.arange(_N, dtype=jnp.int32)
    keys = jnp.concatenate([src * _N + dst, dst * _N + src, ar * _N + ar])
    keys = jnp.sort(keys)
    first = jnp.concatenate(
        [jnp.ones((1,), jnp.bool_), keys[1:] != keys[:-1]])
    row = keys // _N
    col = keys - row * _N
    w = first.astype(jnp.float32)
    return row, col, w


def _seg_sum(vals, row):
    return jax.ops.segment_sum(vals, row, num_segments=_N)


def _ahat_mv(v, row, col, w, dinv):
    coef = w * dinv[row] * dinv[col]
    return _seg_sum(coef[:, None] * v[col], row)


def _att_edge(svec, dvec, h, row, col, w):
    t = svec[row] + dvec[col]
    t = jnp.maximum(t, 0.2 * t)
    coef = w * jnp.exp(t)
    num = _seg_sum(coef[:, None] * h[col], row)
    den = _seg_sum(coef, row)
    return num, den[:, None]


def kernel(x, edge_index, eps, W_hz1q, W_mu, W_ls, W_att_x, a_src_x, a_dst_x,
           W_att_z, a_src_z, a_dst_z, W_y, a_src_y, a_dst_y,
           W_dec_in, W_dec_z1, W_dec_out):
    row, col, w = _edge_list(edge_index)
    deg = _seg_sum(w, row)
    dinv = deg ** -0.5

    xw = _mm(x, W_hz1q)
    hidden = jax.nn.relu(_ahat_mv(xw, row, col, w, dinv))
    hm = _mm(hidden, jnp.concatenate([W_mu, W_ls], axis=1))
    muls = _ahat_mv(hm, row, col, w, dinv)
    mu = muls[:, :_DZ1]
    ls = muls[:, _DZ1:]
    z1q = _z1q(mu, ls, eps)

    z1qt = _padT(z1q)
    dg = _dg_pass(z1q, z1qt)
    t = _mm2(z1q, W_dec_z1, x, W_dec_in)
    hid = _gmat_pass(z1q, z1qt, _padT(dg * t), dg, act=jax.nn.relu)
    u = _mm(hid, W_dec_out)
    emb = _gmat_pass(z1q, z1qt, _padT(dg * u), dg, residual=z1q)
    recon = _recon_pass(emb, _padT(emb)).reshape(-1)

    wcat = jnp.concatenate([W_att_x[0], W_att_x[1]], axis=1)
    hh = _mm(x, wcat)
    h1 = hh[:, :_HY]
    h2 = hh[:, _HY:]
    sd_w = jnp.zeros((2 * _HY, 4), jnp.float32)
    sd_w = sd_w.at[:_HY, 0].set(a_src_x[0]).at[:_HY, 1].set(a_dst_x[0])
    sd_w = sd_w.at[_HY:, 2].set(a_src_x[1]).at[_HY:, 3].set(a_dst_x[1])
    sd = _mm(hh, sd_w)
    n1, d1 = _att_edge(sd[:, 0], sd[:, 1], h1, row, col, w)
    n2, d2 = _att_edge(sd[:, 2], sd[:, 3], h2, row, col, w)
    hx1 = _att_finalize(n1, d1, act=_elu)
    hx2 = _att_finalize(n2, d2, act=_elu)

    hzp = _mm(mu, W_att_z)
    sdz = _mm(hzp, jnp.stack([a_src_z, a_dst_z], axis=1))
    nz, dz = _att_edge(sdz[:, 0], sdz[:, 1], hzp, row, col, w)
    hz = _att_finalize(nz, dz, act=_elu)

    hcat = jnp.concatenate([hx1, hx2, hz], axis=1)
    hy = _mm(hcat, W_y)
    sdy = _mm(hy, jnp.stack([a_src_y, a_dst_y], axis=1))
    ny, dy = _att_edge(sdy[:, 0], sdy[:, 1], hy, row, col, w)
    outputs = _att_finalize(ny, dy)

    return recon, outputs


# speedup vs baseline: 9.0832x; 9.0832x over previous
import functools

import jax
import jax.numpy as jnp
from jax import lax
from jax.experimental import pallas as pl
from jax.experimental.pallas import tpu as pltpu

_N = 10000
_F = 128
_H1 = 32
_DZ1 = 16
_HY = 16
_NH = 2
_NY = 7
_HX = 32
_AR = 0.5

_NPT = 10240
_BR = 400
_BC = 512
_INTERPRET = False


def _ident(v):
    return v


def _mm_kernel(x_ref, w_ref, o_ref, *, act, act_in):
    o_ref[:] = act(
        jnp.dot(act_in(x_ref[:]), w_ref[:],
                preferred_element_type=jnp.float32))


def _mm(x, w, act=_ident, act_in=_ident, br=2000):
    n, k = x.shape
    m = w.shape[1]
    return pl.pallas_call(
        functools.partial(_mm_kernel, act=act, act_in=act_in),
        grid=(n // br,),
        in_specs=[
            pl.BlockSpec((br, k), lambda i: (i, 0)),
            pl.BlockSpec((k, m), lambda i: (0, 0)),
        ],
        out_specs=pl.BlockSpec((br, m), lambda i: (i, 0)),
        out_shape=jax.ShapeDtypeStruct((n, m), jnp.float32),
        interpret=_INTERPRET,
    )(x, w)


def _mm2_kernel(a_ref, wa_ref, b_ref, wb_ref, o_ref):
    o_ref[:] = (
        jnp.dot(a_ref[:], wa_ref[:], preferred_element_type=jnp.float32)
        + jnp.dot(b_ref[:], wb_ref[:], preferred_element_type=jnp.float32))


def _mm2(a, wa, b, wb, br=2000):
    n, ka = a.shape
    kb = b.shape[1]
    m = wa.shape[1]
    return pl.pallas_call(
        _mm2_kernel,
        grid=(n // br,),
        in_specs=[
            pl.BlockSpec((br, ka), lambda i: (i, 0)),
            pl.BlockSpec((ka, m), lambda i: (0, 0)),
            pl.BlockSpec((br, kb), lambda i: (i, 0)),
            pl.BlockSpec((kb, m), lambda i: (0, 0)),
        ],
        out_specs=pl.BlockSpec((br, m), lambda i: (i, 0)),
        out_shape=jax.ShapeDtypeStruct((n, m), jnp.float32),
        interpret=_INTERPRET,
    )(a, wa, b, wb)


def _z1q_kernel(mu_ref, ls_ref, eps_ref, o_ref):
    o_ref[:] = mu_ref[:] + eps_ref[:] * jnp.exp(ls_ref[:])


def _z1q(mu, ls, eps, br=2000):
    n, m = mu.shape
    spec = pl.BlockSpec((br, m), lambda i: (i, 0))
    return pl.pallas_call(
        _z1q_kernel,
        grid=(n // br,),
        in_specs=[spec, spec, spec],
        out_specs=spec,
        out_shape=jax.ShapeDtypeStruct((n, m), jnp.float32),
        interpret=_INTERPRET,
    )(mu, ls, eps)


def _finalize_kernel(num_ref, den_ref, o_ref, *, act):
    o_ref[:] = act(num_ref[:] / den_ref[:])


def _att_finalize(num, den, act=_ident, br=2000):
    n, m = num.shape
    return pl.pallas_call(
        functools.partial(_finalize_kernel, act=act),
        grid=(n // br,),
        in_specs=[
            pl.BlockSpec((br, m), lambda i: (i, 0)),
            pl.BlockSpec((br, 1), lambda i: (i, 0)),
        ],
        out_specs=pl.BlockSpec((br, m), lambda i: (i, 0)),
        out_shape=jax.ShapeDtypeStruct((n, m), jnp.float32),
        interpret=_INTERPRET,
    )(num, den)


def _elu(v):
    return jnp.where(v > 0, v, jnp.exp(v) - 1.0)


def _colmask(c):
    cols = c * _BC + lax.broadcasted_iota(jnp.int32, (1, _BC), 1)
    return (cols < _N).astype(jnp.float32)


def _dg_kernel(z_ref, zt_ref, o_ref):
    def body(c, acc):
        logits = jnp.dot(z_ref[:], zt_ref[:, pl.ds(c * _BC, _BC)],
                         preferred_element_type=jnp.float32)
        g = jax.nn.sigmoid(logits) * _colmask(c)
        return acc + jnp.sum(g, axis=1, keepdims=True)

    s = lax.fori_loop(0, _NPT // _BC, body, jnp.zeros((_BR, 1), jnp.float32))
    o_ref[:] = s ** -0.5


def _dg_pass(z1q, z1qt):
    return pl.pallas_call(
        _dg_kernel,
        grid=(_N // _BR,),
        in_specs=[
            pl.BlockSpec((_BR, _DZ1), lambda i: (i, 0)),
            pl.BlockSpec((_DZ1, _NPT), lambda i: (0, 0)),
        ],
        out_specs=pl.BlockSpec((_BR, 1), lambda i: (i, 0)),
        out_shape=jax.ShapeDtypeStruct((_N, 1), jnp.float32),
        interpret=_INTERPRET,
    )(z1q, z1qt)


def _gmat_kernel(z_ref, zt_ref, vt_ref, dg_ref, o_ref, *, m, act, res_ref=None):
    def body(c, acc):
        logits = jnp.dot(z_ref[:], zt_ref[:, pl.ds(c * _BC, _BC)],
                         preferred_element_type=jnp.float32)
        g = jax.nn.sigmoid(logits) * _colmask(c)
        vchunk = vt_ref[:, pl.ds(c * _BC, _BC)]
        return acc + lax.dot_general(
            g, vchunk, (((1,), (1,)), ((), ())),
            preferred_element_type=jnp.float32)

    acc = lax.fori_loop(0, _NPT // _BC, body, jnp.zeros((_BR, m), jnp.float32))
    out = dg_ref[:] * acc
    if res_ref is not None:
        out = 0.5 * res_ref[:] + 0.5 * out
    o_ref[:] = act(out)


def _gmat_pass(z1q, z1qt, vt, dg, act=_ident, residual=None):
    m = vt.shape[0]
    in_specs = [
        pl.BlockSpec((_BR, _DZ1), lambda i: (i, 0)),
        pl.BlockSpec((_DZ1, _NPT), lambda i: (0, 0)),
        pl.BlockSpec((m, _NPT), lambda i: (0, 0)),
        pl.BlockSpec((_BR, 1), lambda i: (i, 0)),
    ]
    args = [z1q, z1qt, vt, dg]
    if residual is not None:
        in_specs.append(pl.BlockSpec((_BR, m), lambda i: (i, 0)))
        args.append(residual)

        def kern(z, zt, vt_, dg_, res, o):
            _gmat_kernel(z, zt, vt_, dg_, o, m=m, act=act, res_ref=res)
    else:
        def kern(z, zt, vt_, dg_, o):
            _gmat_kernel(z, zt, vt_, dg_, o, m=m, act=act)

    return pl.pallas_call(
        kern,
        grid=(_N // _BR,),
        in_specs=in_specs,
        out_specs=pl.BlockSpec((_BR, m), lambda i: (i, 0)),
        out_shape=jax.ShapeDtypeStruct((_N, m), jnp.float32),
        interpret=_INTERPRET,
    )(*args)


def _recon_kernel(e_ref, et_ref, o_ref):
    o_ref[:] = jnp.dot(e_ref[:], et_ref[:],
                       preferred_element_type=jnp.float32)


def _recon_pass(emb, embt):
    return pl.pallas_call(
        _recon_kernel,
        grid=(_N // _BR, _NPT // _BC),
        in_specs=[
            pl.BlockSpec((_BR, _DZ1), lambda i, j: (i, 0)),
            pl.BlockSpec((_DZ1, _BC), lambda i, j: (0, j)),
        ],
        out_specs=pl.BlockSpec((_BR, _BC), lambda i, j: (i, j)),
        out_shape=jax.ShapeDtypeStruct((_N, _N), jnp.float32),
        interpret=_INTERPRET,
    )(emb, embt)


def _padT(v):
    return jnp.pad(v, ((0, _NPT - _N), (0, 0))).T


_E2 = 2 * 160000 + _N
_NW = 32
_CB = 256
_NCH = -(-_E2 // (_NW * _CB))
_EW = _NCH * _CB
_EP = _NW * _EW
_AW = 32


def _edge_pass_kernel(row3, col3, w3, svec, dvec, hmat, zeros,
                      out, acc, rowj, colj, wj, sv, dv, hbuf,
                      wide, coefv, gsem, *, mode):
    from jax.experimental.pallas import tpu_sc as plsc

    cid = lax.axis_index("c")
    sid = lax.axis_index("s")
    wid = sid * 2 + cid

    pltpu.sync_copy(row3.at[wid], rowj)
    pltpu.sync_copy(col3.at[wid], colj)
    pltpu.sync_copy(w3.at[wid], wj)
    pltpu.sync_copy(svec, sv)
    pltpu.sync_copy(dvec, dv)

    @pl.when(sid == 0)
    def _():
        pltpu.sync_copy(zeros, acc)

    plsc.subcore_barrier()

    def chunk(j, carry):
        for g in range(_CB // 16):
            r16 = rowj[j, pl.ds(g * 16, 16)]
            c16 = colj[j, pl.ds(g * 16, 16)]
            w16 = wj[j, pl.ds(g * 16, 16)]
            s16 = plsc.load_gather(sv, [r16])
            d16 = plsc.load_gather(dv, [c16])
            if mode == "att":
                t = s16 + d16
                t = jnp.maximum(t, 0.2 * t)
                coef = w16 * jnp.exp(t)
            else:
                coef = w16 * s16 * d16
            coefv[pl.ds(g * 16, 16)] = coef

        pltpu.async_copy(hmat.at[colj.at[j]], hbuf, gsem).wait()

        def scale(r, c2):
            cs = plsc.load_gather(coefv, [jnp.full((16,), r, jnp.int32)])
            wide[r, 0:16] = hbuf[r, 0:16] * cs
            wide[r, 16:32] = hbuf[r, 16:32] * cs
            return c2

        lax.fori_loop(0, _CB, scale, 0)

        pltpu.sync_copy(wide, acc.at[rowj.at[j]], add=True)
        return carry

    lax.fori_loop(0, _NCH, chunk, 0)
    plsc.subcore_barrier()

    @pl.when(sid == 0)
    def _():
        pltpu.sync_copy(acc, out.at[cid])


def _edge_pass(row3, col3, w3, svec, dvec, hmat, mode):
    from jax.experimental.pallas import tpu_sc as plsc

    zeros = jnp.zeros((_N, _AW), jnp.float32)
    mesh = plsc.VectorSubcoreMesh(core_axis_name="c", subcore_axis_name="s")
    kern = pl.kernel(
        functools.partial(_edge_pass_kernel, mode=mode),
        mesh=mesh,
        compiler_params=pltpu.CompilerParams(
            needs_layout_passes=False, use_tc_tiling_on_sc=False),
        out_type=jax.ShapeDtypeStruct((2, _N, _AW), jnp.float32),
        scratch_types=[
            pltpu.VMEM_SHARED((_N, _AW), jnp.float32),
            pltpu.VMEM((_NCH, _CB), jnp.int32),
            pltpu.VMEM((_NCH, _CB), jnp.int32),
            pltpu.VMEM((_NCH, _CB), jnp.float32),
            pltpu.VMEM((_N,), jnp.float32),
            pltpu.VMEM((_N,), jnp.float32),
            pltpu.VMEM((_CB, _AW), jnp.float32),
            pltpu.VMEM((_CB, _AW), jnp.float32),
            pltpu.VMEM((_CB,), jnp.float32),
            pltpu.SemaphoreType.DMA,
        ],
    )
    parts = kern(row3, col3, w3, svec, dvec, hmat, zeros)
    return parts[0] + parts[1]


def _edge_list(edge_index):
    src = edge_index[0].astype(jnp.int32)
    dst = edge_index[1].astype(jnp.int32)
    ar = jnp.arange(_N, dtype=jnp.int32)
    keys = jnp.concatenate([src * _N + dst, dst * _N + src, ar * _N + ar])
    keys = jnp.sort(keys)
    first = jnp.concatenate(
        [jnp.ones((1,), jnp.bool_), keys[1:] != keys[:-1]])
    row = keys // _N
    col = keys - row * _N
    w = first.astype(jnp.float32)
    pad = _EP - _E2
    row3 = jnp.pad(row, (0, pad)).reshape(_NW, _NCH, _CB)
    col3 = jnp.pad(col, (0, pad)).reshape(_NW, _NCH, _CB)
    w3 = jnp.pad(w, (0, pad)).reshape(_NW, _NCH, _CB)
    return row3, col3, w3


def kernel(x, edge_index, eps, W_hz1q, W_mu, W_ls, W_att_x, a_src_x, a_dst_x,
           W_att_z, a_src_z, a_dst_z, W_y, a_src_y, a_dst_y,
           W_dec_in, W_dec_z1, W_dec_out):
    row3, col3, w3 = _edge_list(edge_index)
    ones = jnp.ones((_N,), jnp.float32)

    def edge_pass(svec, dvec, hmat, mode):
        return _edge_pass(row3, col3, w3, svec, dvec, hmat, mode)

    def att_pass(svec, dvec, h16):
        hmat = jnp.concatenate(
            [h16, jnp.ones((_N, 1), jnp.float32),
             jnp.zeros((_N, _AW - h16.shape[1] - 1), jnp.float32)], axis=1)
        acc = edge_pass(svec, dvec, hmat, "att")
        return acc[:, :h16.shape[1]], acc[:, h16.shape[1]:h16.shape[1] + 1]

    hone = jnp.concatenate(
        [jnp.ones((_N, 1), jnp.float32),
         jnp.zeros((_N, _AW - 1), jnp.float32)], axis=1)
    deg = edge_pass(ones, ones, hone, "mul")[:, 0]
    dinv = deg ** -0.5

    xw = _mm(x, W_hz1q)
    hidden = edge_pass(dinv, dinv, xw, "mul")
    hm = _mm(hidden, jnp.concatenate([W_mu, W_ls], axis=1),
             act_in=jax.nn.relu)
    muls = edge_pass(dinv, dinv, hm, "mul")
    mu = muls[:, :_DZ1]
    ls = muls[:, _DZ1:]
    z1q = _z1q(mu, ls, eps)

    z1qt = _padT(z1q)
    dg = _dg_pass(z1q, z1qt)
    t = _mm2(z1q, W_dec_z1, x, W_dec_in)
    hid = _gmat_pass(z1q, z1qt, _padT(dg * t), dg, act=jax.nn.relu)
    u = _mm(hid, W_dec_out)
    emb = _gmat_pass(z1q, z1qt, _padT(dg * u), dg, residual=z1q)
    recon = _recon_pass(emb, _padT(emb)).reshape(-1)

    wcat = jnp.concatenate([W_att_x[0], W_att_x[1]], axis=1)
    hh = _mm(x, wcat)
    h1 = hh[:, :_HY]
    h2 = hh[:, _HY:]
    sd_w = jnp.zeros((2 * _HY, 4), jnp.float32)
    sd_w = sd_w.at[:_HY, 0].set(a_src_x[0]).at[:_HY, 1].set(a_dst_x[0])
    sd_w = sd_w.at[_HY:, 2].set(a_src_x[1]).at[_HY:, 3].set(a_dst_x[1])
    sd = _mm(hh, sd_w)
    n1, d1 = att_pass(sd[:, 0], sd[:, 1], h1)
    n2, d2 = att_pass(sd[:, 2], sd[:, 3], h2)
    hx1 = _att_finalize(n1, d1, act=_elu)
    hx2 = _att_finalize(n2, d2, act=_elu)

    hzp = _mm(mu, W_att_z)
    sdz = _mm(hzp, jnp.stack([a_src_z, a_dst_z], axis=1))
    nz, dz = att_pass(sdz[:, 0], sdz[:, 1], hzp)
    hz = _att_finalize(nz, dz, act=_elu)

    hcat = jnp.concatenate([hx1, hx2, hz], axis=1)
    hy = _mm(hcat, W_y)
    sdy = _mm(hy, jnp.stack([a_src_y, a_dst_y], axis=1))
    ny, dy = att_pass(sdy[:, 0], sdy[:, 1], hy)
    outputs = _att_finalize(ny, dy)

    return recon, outputs
